# baseline (speedup 1.0000x reference)
import functools

import jax
import jax.numpy as jnp
from jax import lax
from jax.experimental import pallas as pl
from jax.experimental.pallas import tpu as pltpu
from jax.experimental.pallas import tpu_sc as plsc

_B = 4096 * 50
_D = 64
_NC = 2
_NS = 16
_NW = _NC * _NS
_BPW = _B // _NW
_CHUNK = 128
_NCH = _BPW // _CHUNK
_SCALE = 8.0

_mesh = plsc.VectorSubcoreMesh(
    core_axis_name="c", subcore_axis_name="s", num_cores=_NC, num_subcores=_NS
)


@functools.partial(
    pl.kernel,
    out_type=jax.ShapeDtypeStruct((_B, _D), jnp.float32),
    mesh=_mesh,
    scratch_types=[
        pltpu.VMEM((_NCH, _CHUNK), jnp.int32),
        pltpu.VMEM((_CHUNK, _D), jnp.float32),
        pltpu.VMEM((_CHUNK, _D), jnp.float32),
        pltpu.SemaphoreType.DMA,
        pltpu.SemaphoreType.DMA,
    ],
    compiler_params=pltpu.CompilerParams(use_tc_tiling_on_sc=False),
)
def _emb_lookup(tokens_hbm, table_hbm, out_hbm, idx_v, rows0, rows1, sem0, sem1):
    wid = lax.axis_index("s") * _NC + lax.axis_index("c")
    base = wid * _BPW
    pltpu.sync_copy(tokens_hbm.at[wid], idx_v)

    bufs = (rows0, rows1)
    sems = (sem0, sem1)

    def issue_gather(c):
        return pltpu.async_copy(
            table_hbm.at[idx_v.at[c]], bufs[c % 2], sems[c % 2]
        )

    handles = [None] * _NCH
    handles[0] = issue_gather(0)
    for c in range(_NCH):
        if c + 1 < _NCH:
            handles[c + 1] = issue_gather(c + 1)
        handles[c].wait()
        buf = bufs[c % 2]

        def _scale_row(r, carry, buf=buf):
            for j in range(_D // 16):
                sl = pl.ds(j * 16, 16)
                buf[r, sl] = buf[r, sl] * _SCALE
            return carry

        lax.fori_loop(0, _CHUNK, _scale_row, 0)

        pltpu.sync_copy(buf, out_hbm.at[pl.ds(base + c * _CHUNK, _CHUNK)])


def kernel(b_tokens, table):
    tokens = b_tokens.reshape(_NW, _NCH, _CHUNK).astype(jnp.int32)
    out = _emb_lookup(tokens, table)
    return out.reshape(b_tokens.shape[0], b_tokens.shape[1], _D)
